# Initial kernel scaffold; baseline (speedup 1.0000x reference)
import jax
import jax.numpy as jnp
from jax.experimental import pallas as pl


def kernel(x, enc_W, enc_b, W1, b1, g1, be1, W2, b2, g2, be2, rW1, rb1, rW2, rb2, c_2, u_2, batch):
    raise NotImplementedError("write your pallas kernel here")


# speedup vs baseline: 3.6866x; 3.6866x over previous
import functools

import jax
import jax.numpy as jnp
from jax import lax
from jax.experimental import pallas as pl
from jax.experimental.pallas import tpu as pltpu
from jax.experimental.pallas import tpu_sc as plsc

_N = 10000
_D = 128
_E = 320000
_G = 64
_C = 40
_NC = 2
_NS = 16
_NW = _NC * _NS
_CK = 128
_PER_W = -(-(_E // _NW) // _CK) * _CK
_NCHUNK = _PER_W // _CK
_E_PAD = _PER_W * _NW
_ACC_ROWS = _N + 8
_RPT = (_N // _NS) // 8 * 8
_TAIL = _N - _RPT * _NS


def _agg_body(h_hbm, c_hbm, u_hbm, out_hbm, cidx, uidx, rows, acc, sem):
    ci = lax.axis_index("c")
    si = lax.axis_index("s")
    wid = si * _NC + ci
    r0 = si * _RPT
    pltpu.sync_copy(h_hbm.at[pl.ds(r0, _RPT)], acc.at[pl.ds(r0, _RPT)])

    @pl.when(si == _NS - 1)
    def _init_tail():
        pltpu.sync_copy(h_hbm.at[pl.ds(_RPT * _NS, _TAIL)],
                        acc.at[pl.ds(_RPT * _NS, _TAIL)])

    plsc.subcore_barrier()

    base = wid * _PER_W

    def body(j, carry):
        off = base + j * _CK
        pltpu.sync_copy(c_hbm.at[pl.ds(off, _CK)], cidx)
        pltpu.sync_copy(u_hbm.at[pl.ds(off, _CK)], uidx)
        pltpu.async_copy(h_hbm.at[cidx], rows, sem).wait()
        pltpu.sync_copy(rows, acc.at[uidx], add=True)
        return carry

    lax.fori_loop(0, _NCHUNK, body, 0)

    plsc.subcore_barrier()
    pltpu.sync_copy(acc.at[pl.ds(r0, _RPT)],
                    out_hbm.at[pl.ds(ci * _N + r0, _RPT)])

    @pl.when(si == _NS - 1)
    def _out_tail():
        pltpu.sync_copy(acc.at[pl.ds(_RPT * _NS, _TAIL)],
                        out_hbm.at[pl.ds(ci * _N + _RPT * _NS, _TAIL)])


_agg = pl.kernel(
    _agg_body,
    out_type=jax.ShapeDtypeStruct((_NC * _N, _D), jnp.float32),
    mesh=plsc.VectorSubcoreMesh(core_axis_name="c", subcore_axis_name="s"),
    scratch_types=[
        pltpu.VMEM((_CK,), jnp.int32),
        pltpu.VMEM((_CK,), jnp.int32),
        pltpu.VMEM((_CK, _D), jnp.float32),
        pltpu.VMEM_SHARED((_ACC_ROWS, _D), jnp.float32),
        pltpu.SemaphoreType.DMA,
    ],
)


def _bn(a, g, b):
    mu = jnp.mean(a, axis=0, keepdims=True)
    v = jnp.mean((a - mu) ** 2, axis=0, keepdims=True)
    return g * (a - mu) / jnp.sqrt(v + 1e-5) + b


def _enc_body(x_ref, w_ref, b_ref, o_ref):
    o_ref[...] = (
        jnp.dot(x_ref[...], w_ref[...], preferred_element_type=jnp.float32)
        + b_ref[...]
    )


_enc = pl.pallas_call(
    _enc_body, out_shape=jax.ShapeDtypeStruct((_N, _D), jnp.float32)
)


def _layer_h(p_ref, h_ref, w1_ref, b1_ref, g1_ref, be1_ref, w2_ref, b2_ref,
             g2_ref, be2_ref):
    m = p_ref[0:_N, :] + p_ref[_N:2 * _N, :] - h_ref[...]
    a = jnp.dot(m, w1_ref[...], preferred_element_type=jnp.float32) + b1_ref[...]
    a = jnp.maximum(_bn(a, g1_ref[...], be1_ref[...]), 0.0)
    z = jnp.dot(a, w2_ref[...], preferred_element_type=jnp.float32) + b2_ref[...]
    return jnp.maximum(_bn(z, g2_ref[...], be2_ref[...]), 0.0)


def _mlp_body(p_ref, h_ref, w1_ref, b1_ref, g1_ref, be1_ref, w2_ref, b2_ref,
              g2_ref, be2_ref, o_ref):
    o_ref[...] = _layer_h(p_ref, h_ref, w1_ref, b1_ref, g1_ref, be1_ref,
                          w2_ref, b2_ref, g2_ref, be2_ref)


_mlp = pl.pallas_call(
    _mlp_body, out_shape=jax.ShapeDtypeStruct((_N, _D), jnp.float32)
)


def _mlp3_body(p_ref, h_ref, w1_ref, b1_ref, g1_ref, be1_ref, w2_ref, b2_ref,
               g2_ref, be2_ref, batch_ref, rw1_ref, rb1_ref, rw2_ref, rb2_ref,
               o_ref):
    h3 = _layer_h(p_ref, h_ref, w1_ref, b1_ref, g1_ref, be1_ref,
                  w2_ref, b2_ref, g2_ref, be2_ref)
    onehot = (lax.broadcasted_iota(jnp.int32, (_G, 1), 0)
              == batch_ref[...]).astype(jnp.float32)
    pooled = jnp.dot(onehot, h3, preferred_element_type=jnp.float32)
    r = jnp.maximum(
        jnp.dot(pooled, rw1_ref[...], preferred_element_type=jnp.float32)
        + rb1_ref[...], 0.0)
    o_ref[...] = (
        jnp.dot(r, rw2_ref[...], preferred_element_type=jnp.float32)
        + rb2_ref[...]
    )


_mlp3 = pl.pallas_call(
    _mlp3_body, out_shape=jax.ShapeDtypeStruct((_G, _C), jnp.float32)
)


def kernel(x, enc_W, enc_b, W1, b1, g1, be1, W2, b2, g2, be2,
           rW1, rb1, rW2, rb2, c_2, u_2, batch):
    pad = _E_PAD - _E
    c_p = jnp.concatenate([c_2.astype(jnp.int32),
                           jnp.zeros((pad,), jnp.int32)])
    u_p = jnp.concatenate([u_2.astype(jnp.int32),
                           jnp.full((pad,), _N, jnp.int32)])
    h = _enc(x, enc_W, enc_b.reshape(1, _D))
    out = None
    for i in range(3):
        p = _agg(h, c_p, u_p)
        args = (p, h, W1[i], b1[i].reshape(1, -1), g1[i].reshape(1, -1),
                be1[i].reshape(1, -1), W2[i], b2[i].reshape(1, -1),
                g2[i].reshape(1, -1), be2[i].reshape(1, -1))
        if i < 2:
            h = _mlp(*args)
        else:
            out = _mlp3(*args, batch.reshape(1, _N).astype(jnp.int32),
                        rW1, rb1.reshape(1, -1), rW2, rb2.reshape(1, -1))
    return out
